# Initial kernel scaffold; baseline (speedup 1.0000x reference)
import jax
import jax.numpy as jnp
from jax.experimental import pallas as pl


def kernel(head, rel, edge_src, edge_rel, edge_dst, ent_emb, query_emb, rel_embs, W_ent, b_ent, W_lin, b_lin, ln_g, ln_b, W_cls, b_cls):
    raise NotImplementedError("write your pallas kernel here")


# speedup vs baseline: 4.9231x; 4.9231x over previous
import functools

import jax
import jax.numpy as jnp
import numpy as np
from jax import lax
from jax.experimental import pallas as pl
from jax.experimental.pallas import tpu as pltpu
from jax.experimental.pallas import tpu_sc as plsc

N_ENT = 10000
N_REL = 41
H = 128
N_EDGES = 160000
B = 4
DEGREE = 3

NW = 16
E_PAD = 163840
EPW = E_PAD // NW
K = 128
N_PAD = 10240
NACC = N_PAD
BN = 320
NBLK = N_PAD // BN
C0 = float(1.0 / np.sqrt(H) - 0.5)

_mesh = plsc.VectorSubcoreMesh(core_axis_name="c", subcore_axis_name="s")


def _iota16():
    return lax.broadcasted_iota(jnp.int32, (16,), 0)


KD = E_PAD // 32


@functools.partial(
    pl.kernel,
    out_type=jax.ShapeDtypeStruct((2, N_PAD, 16), jnp.float32),
    mesh=_mesh,
    compiler_params=pltpu.CompilerParams(use_tc_tiling_on_sc=False),
    scratch_types=[
        pltpu.VMEM_SHARED((NACC, 16), jnp.float32),
        pltpu.VMEM((KD,), jnp.int32),
        pltpu.VMEM((KD,), jnp.int32),
        pltpu.VMEM((KD, 16), jnp.float32),
        pltpu.VMEM((64, 16), jnp.float32),
    ],
)
def _degree_sc(er, ed, out, acc, relb, dstb, rowsb, zbuf):
    c = lax.axis_index("c")
    s = lax.axis_index("s")
    wid = c * NW + s

    @pl.loop(0, 64)
    def _(i):
        zbuf[i, pl.ds(0, 16)] = jnp.zeros((16,), jnp.float32)

    for z in range(10):
        pltpu.sync_copy(zbuf, acc.at[pl.ds(s * 640 + z * 64, 64)])
    plsc.subcore_barrier()

    base = wid * KD
    pltpu.sync_copy(er.at[pl.ds(base, KD)], relb)
    pltpu.sync_copy(ed.at[pl.ds(base, KD)], dstb)

    @pl.loop(0, KD // 16)
    def _(i):
        r16 = relb[pl.ds(i * 16, 16)]
        v16 = jnp.where((r16 >= 1) & (r16 < (N_REL - 1) // 2),
                        jnp.float32(1.0), jnp.float32(0.0))
        for j in range(16):
            rowsb[i * 16 + j, pl.ds(0, 16)] = jnp.broadcast_to(v16[j], (16,))

    pltpu.sync_copy(rowsb, acc.at[dstb], add=True)
    plsc.subcore_barrier()

    @pl.loop(0, 16)
    def _(z):
        pltpu.sync_copy(acc.at[pl.ds(s * 640 + z * 40, 40)],
                        out.at[c, pl.ds(s * 640 + z * 40, 40), :])


@functools.partial(
    pl.kernel,
    out_type=jax.ShapeDtypeStruct((B * N_PAD, H), jnp.float32),
    mesh=_mesh,
    compiler_params=pltpu.CompilerParams(use_tc_tiling_on_sc=False),
    scratch_types=[
        pltpu.VMEM_SHARED((NACC, H), jnp.float32),
        pltpu.VMEM((K,), jnp.int32),
        pltpu.VMEM((K,), jnp.int32),
        pltpu.VMEM((K,), jnp.int32),
        pltpu.VMEM((K, H), jnp.float32),
        pltpu.VMEM((K, H), jnp.float32),
        pltpu.VMEM((32, H), jnp.float32),
        pltpu.SemaphoreType.DMA,
        pltpu.SemaphoreType.DMA,
    ],
)
def _rspmm_sc(es, er, ed, rel_t, xe, out, acc, srcb, relb, dstb, rows,
              rrows, zbuf, sem0, sem1):
    c = lax.axis_index("c")
    s = lax.axis_index("s")

    @pl.loop(0, 32)
    def _(i):
        for h in range(H // 16):
            zbuf[i, pl.ds(h * 16, 16)] = jnp.zeros((16,), jnp.float32)

    for b in range(2):
        bidx = c * 2 + b

        for z in range(20):
            pltpu.sync_copy(zbuf, acc.at[pl.ds(s * 640 + z * 32, 32)])

        plsc.subcore_barrier()

        row_off = bidx * N_PAD

        @pl.loop(0, EPW // K)
        def _(ch):
            ebase = s * EPW + ch * K
            pltpu.sync_copy(es.at[pl.ds(ebase, K)], srcb)
            pltpu.sync_copy(er.at[pl.ds(ebase, K)], relb)
            pltpu.sync_copy(ed.at[pl.ds(ebase, K)], dstb)

            @pl.loop(0, K // 16)
            def _(i):
                srcb[pl.ds(i * 16, 16)] = srcb[pl.ds(i * 16, 16)] + row_off

            cp0 = pltpu.async_copy(xe.at[srcb], rows, sem0)
            cp1 = pltpu.async_copy(rel_t.at[relb], rrows, sem1)
            cp0.wait()
            cp1.wait()

            @pl.loop(0, K)
            def _(k):
                for h in range(H // 16):
                    rows[k, pl.ds(h * 16, 16)] = (
                        rows[k, pl.ds(h * 16, 16)]
                        * rrows[k, pl.ds(h * 16, 16)]
                    )

            pltpu.sync_copy(rows, acc.at[dstb], add=True)

        plsc.subcore_barrier()

        @pl.loop(0, 16)
        def _(z):
            pltpu.sync_copy(acc.at[pl.ds(s * 640 + z * 40, 40)],
                            out.at[pl.ds(row_off + s * 640 + z * 40, 40)])

        plsc.subcore_barrier()


def _tc_spec(idx_map, shape):
    return pl.BlockSpec(shape, idx_map)


def _init_mm_body(head_ref, degp_ref, ent_ref, q_ref, w_ref, bias_ref,
                  out_ref):
    b = pl.program_id(0)
    i = pl.program_id(1)
    deg = (degp_ref[0] + degp_ref[1])[:, 0:1]
    hid = jnp.where(deg >= DEGREE, ent_ref[...],
                    jnp.float32(C0))
    nglob = i * BN + lax.broadcasted_iota(jnp.int32, (BN, 1), 0)
    is_head = nglob == head_ref[b]
    hid = jnp.where(is_head, q_ref[...], hid)
    out_ref[...] = (
        jnp.dot(hid, w_ref[...], preferred_element_type=jnp.float32)
        + bias_ref[...]
    )


def _init_mm(head, degp, ent_emb, query_emb, w, bias):
    return pl.pallas_call(
        _init_mm_body,
        grid=(B, NBLK),
        in_specs=[
            pl.BlockSpec(memory_space=pltpu.SMEM),
            _tc_spec(lambda b, i: (0, i, 0), (2, BN, 16)),
            _tc_spec(lambda b, i: (i, 0), (BN, H)),
            _tc_spec(lambda b, i: (0, 0), (1, H)),
            _tc_spec(lambda b, i: (0, 0), (H, H)),
            _tc_spec(lambda b, i: (0, 0), (1, H)),
        ],
        out_specs=_tc_spec(lambda b, i: (b * NBLK + i, 0), (BN, H)),
        out_shape=jax.ShapeDtypeStruct((B * N_PAD, H), jnp.float32),
    )(head, degp, ent_emb, query_emb, w, bias)


def _post_layer(upd, xe, wl, bl, g, beta, eps=1e-5):
    t = jnp.dot(upd, wl, preferred_element_type=jnp.float32) + bl + xe
    mu = jnp.mean(t, axis=-1, keepdims=True)
    var = jnp.mean((t - mu) ** 2, axis=-1, keepdims=True)
    return jnp.maximum(g * (t - mu) / jnp.sqrt(var + eps) + beta, 0.0)


def _mid_mm_body(upd_ref, xe_ref, wl_ref, bl_ref, g_ref, beta_ref, w2_ref,
                 b2_ref, out_ref):
    y = _post_layer(upd_ref[...], xe_ref[...], wl_ref[...], bl_ref[...],
                    g_ref[...], beta_ref[...])
    out_ref[...] = (
        jnp.dot(y, w2_ref[...], preferred_element_type=jnp.float32)
        + b2_ref[...]
    )


def _mid_mm(upd, xe, wl, bl, g, beta, w2, b2):
    blk = _tc_spec(lambda b, i: (b * NBLK + i, 0), (BN, H))
    par = _tc_spec(lambda b, i: (0, 0), (1, H))
    return pl.pallas_call(
        _mid_mm_body,
        grid=(B, NBLK),
        in_specs=[blk, blk, _tc_spec(lambda b, i: (0, 0), (H, H)), par, par,
                  par, _tc_spec(lambda b, i: (0, 0), (H, H)), par],
        out_specs=blk,
        out_shape=jax.ShapeDtypeStruct((B * N_PAD, H), jnp.float32),
    )(upd, xe, wl, bl, g, beta, w2, b2)


def _final_body(upd_ref, xe_ref, wl_ref, bl_ref, g_ref, beta_ref, wc_ref,
                bc_ref, out_ref):
    y = _post_layer(upd_ref[...], xe_ref[...], wl_ref[...], bl_ref[...],
                    g_ref[...], beta_ref[...])
    out_ref[...] = (
        jnp.dot(y, wc_ref[...], preferred_element_type=jnp.float32)
        + bc_ref[...]
    )


def _final_mm(upd, xe, wl, bl, g, beta, wc_pad, bc):
    blk = _tc_spec(lambda b, i: (b * NBLK + i, 0), (BN, H))
    par = _tc_spec(lambda b, i: (0, 0), (1, H))
    return pl.pallas_call(
        _final_body,
        grid=(B, NBLK),
        in_specs=[blk, blk, _tc_spec(lambda b, i: (0, 0), (H, H)), par, par,
                  par, _tc_spec(lambda b, i: (0, 0), (H, H)), par],
        out_specs=blk,
        out_shape=jax.ShapeDtypeStruct((B * N_PAD, H), jnp.float32),
    )(upd, xe, wl, bl, g, beta, wc_pad, bc)


def kernel(head, rel, edge_src, edge_rel, edge_dst, ent_emb, query_emb,
           rel_embs, W_ent, b_ent, W_lin, b_lin, ln_g, ln_b, W_cls, b_cls):
    npad = E_PAD - N_EDGES
    es = jnp.concatenate([edge_src, jnp.zeros((npad,), jnp.int32)])
    er = jnp.concatenate([edge_rel, jnp.zeros((npad,), jnp.int32)])
    ed = jnp.concatenate([edge_dst, jnp.full((npad,), N_ENT, jnp.int32)])

    degp = _degree_sc(er, ed)

    ent_p = jnp.concatenate(
        [ent_emb, jnp.zeros((N_PAD - N_ENT, H), jnp.float32)])
    xe = _init_mm(head, degp, ent_p, query_emb, W_ent[0],
                  b_ent[0].reshape(1, H))
    upd = _rspmm_sc(es, er, ed, rel_embs[0], xe)
    xe = _mid_mm(upd, xe, W_lin[0], b_lin[0].reshape(1, H),
                 ln_g[0].reshape(1, H), ln_b[0].reshape(1, H),
                 W_ent[1], b_ent[1].reshape(1, H))
    upd = _rspmm_sc(es, er, ed, rel_embs[1], xe)

    wc_pad = jnp.zeros((H, H), jnp.float32).at[:, 0].set(W_cls[:, 0])
    bc = jnp.zeros((1, H), jnp.float32).at[0, 0].set(b_cls[0])
    outf = _final_mm(upd, xe, W_lin[1], b_lin[1].reshape(1, H),
                     ln_g[1].reshape(1, H), ln_b[1].reshape(1, H),
                     wc_pad, bc)
    return outf.reshape(B, N_PAD, H)[:, :N_ENT, 0]


# speedup vs baseline: 5.1504x; 1.0462x over previous
import functools

import jax
import jax.numpy as jnp
import numpy as np
from jax import lax
from jax.experimental import pallas as pl
from jax.experimental.pallas import tpu as pltpu
from jax.experimental.pallas import tpu_sc as plsc

N_ENT = 10000
N_REL = 41
H = 128
N_EDGES = 160000
B = 4
DEGREE = 3

NW = 16
E_PAD = 163840
EPW = E_PAD // NW
K = 80
NCH = 10240 // K
N_PAD = 10240
NACC = N_PAD
BN = 320
NBLK = N_PAD // BN
C0 = float(1.0 / np.sqrt(H) - 0.5)

_mesh = plsc.VectorSubcoreMesh(core_axis_name="c", subcore_axis_name="s")


def _iota16():
    return lax.broadcasted_iota(jnp.int32, (16,), 0)


KD = E_PAD // 32


@functools.partial(
    pl.kernel,
    out_type=jax.ShapeDtypeStruct((2, N_PAD, 16), jnp.float32),
    mesh=_mesh,
    compiler_params=pltpu.CompilerParams(use_tc_tiling_on_sc=False),
    scratch_types=[
        pltpu.VMEM_SHARED((NACC, 16), jnp.float32),
        pltpu.VMEM((KD,), jnp.int32),
        pltpu.VMEM((KD,), jnp.int32),
        pltpu.VMEM((KD, 16), jnp.float32),
        pltpu.VMEM((64, 16), jnp.float32),
    ],
)
def _degree_sc(er, ed, out, acc, relb, dstb, rowsb, zbuf):
    c = lax.axis_index("c")
    s = lax.axis_index("s")
    wid = c * NW + s

    @pl.loop(0, 64)
    def _(i):
        zbuf[i, pl.ds(0, 16)] = jnp.zeros((16,), jnp.float32)

    for z in range(10):
        pltpu.sync_copy(zbuf, acc.at[pl.ds(s * 640 + z * 64, 64)])
    plsc.subcore_barrier()

    base = wid * KD
    pltpu.sync_copy(er.at[pl.ds(base, KD)], relb)
    pltpu.sync_copy(ed.at[pl.ds(base, KD)], dstb)

    @pl.loop(0, KD // 16)
    def _(i):
        r16 = relb[pl.ds(i * 16, 16)]
        v16 = jnp.where((r16 >= 1) & (r16 < (N_REL - 1) // 2),
                        jnp.float32(1.0), jnp.float32(0.0))
        for j in range(16):
            rowsb[i * 16 + j, pl.ds(0, 16)] = jnp.broadcast_to(v16[j], (16,))

    pltpu.sync_copy(rowsb, acc.at[dstb], add=True)
    plsc.subcore_barrier()

    @pl.loop(0, 16)
    def _(z):
        pltpu.sync_copy(acc.at[pl.ds(s * 640 + z * 40, 40)],
                        out.at[c, pl.ds(s * 640 + z * 40, 40), :])


@functools.partial(
    pl.kernel,
    out_type=jax.ShapeDtypeStruct((B * N_PAD, H), jnp.float32),
    mesh=_mesh,
    compiler_params=pltpu.CompilerParams(use_tc_tiling_on_sc=False),
    scratch_types=[
        pltpu.VMEM_SHARED((NACC, H), jnp.float32),
        pltpu.VMEM((K,), jnp.int32),
        pltpu.VMEM((K,), jnp.int32),
        pltpu.VMEM((K,), jnp.int32),
        pltpu.VMEM((K,), jnp.int32),
        pltpu.VMEM((K,), jnp.int32),
        pltpu.VMEM((K,), jnp.int32),
        pltpu.VMEM((K, H), jnp.float32),
        pltpu.VMEM((K, H), jnp.float32),
        pltpu.VMEM((K, H), jnp.float32),
        pltpu.VMEM((K, H), jnp.float32),
        pltpu.VMEM((16, H), jnp.float32),
        pltpu.SemaphoreType.DMA,
        pltpu.SemaphoreType.DMA,
        pltpu.SemaphoreType.DMA,
        pltpu.SemaphoreType.DMA,
        pltpu.SemaphoreType.DMA,
        pltpu.SemaphoreType.DMA,
    ],
)
def _rspmm_sc(es, er, ed, rel_t, xe, out, acc, srcb0, srcb1, relb0, relb1,
              dstb0, dstb1, rows0, rows1, rrows0, rrows1, zbuf,
              sg0, sg1, sq0, sq1, ss0, ss1):
    c = lax.axis_index("c")
    s = lax.axis_index("s")
    srcb = (srcb0, srcb1)
    relb = (relb0, relb1)
    dstb = (dstb0, dstb1)
    rows = (rows0, rows1)
    rrows = (rrows0, rrows1)
    sg = (sg0, sg1)
    sq = (sq0, sq1)
    ss = (ss0, ss1)

    @pl.loop(0, 16)
    def _(i):
        for h in range(H // 16):
            zbuf[i, pl.ds(h * 16, 16)] = jnp.zeros((16,), jnp.float32)

    for b in range(2):
        bidx = c * 2 + b
        row_off = bidx * N_PAD

        for z in range(40):
            pltpu.sync_copy(zbuf, acc.at[pl.ds(s * 640 + z * 16, 16)])

        plsc.subcore_barrier()

        def load_and_gather(i, ch):
            ebase = s * EPW + ch * K
            pltpu.sync_copy(es.at[pl.ds(ebase, K)], srcb[i])
            pltpu.sync_copy(er.at[pl.ds(ebase, K)], relb[i])
            pltpu.sync_copy(ed.at[pl.ds(ebase, K)], dstb[i])

            @pl.loop(0, K // 16)
            def _(j):
                srcb[i][pl.ds(j * 16, 16)] = (
                    srcb[i][pl.ds(j * 16, 16)] + row_off)

            pltpu.async_copy(xe.at[srcb[i]], rows[i], sg[i])
            pltpu.async_copy(rel_t.at[relb[i]], rrows[i], sq[i])

        def wait_gathers(i):
            pltpu.make_async_copy(xe.at[srcb[i]], rows[i], sg[i]).wait()
            pltpu.make_async_copy(rel_t.at[relb[i]], rrows[i], sq[i]).wait()

        def multiply(i):
            @pl.loop(0, K)
            def _(k):
                for h in range(H // 16):
                    rows[i][k, pl.ds(h * 16, 16)] = (
                        rows[i][k, pl.ds(h * 16, 16)]
                        * rrows[i][k, pl.ds(h * 16, 16)]
                    )

        load_and_gather(0, 0)
        load_and_gather(1, 1)

        @pl.loop(0, NCH // 2)
        def _(gg):
            wait_gathers(0)
            multiply(0)
            pltpu.async_copy(rows[0], acc.at[dstb[0]], ss[0], add=True)
            wait_gathers(1)
            multiply(1)
            pltpu.async_copy(rows[1], acc.at[dstb[1]], ss[1], add=True)
            pltpu.make_async_copy(rows[0], acc.at[dstb[0]], ss[0]).wait()

            @pl.when(gg < NCH // 2 - 1)
            def _():
                load_and_gather(0, 2 * gg + 2)

            pltpu.make_async_copy(rows[1], acc.at[dstb[1]], ss[1]).wait()

            @pl.when(gg < NCH // 2 - 1)
            def _():
                load_and_gather(1, 2 * gg + 3)

        plsc.subcore_barrier()

        @pl.loop(0, 16)
        def _(z):
            pltpu.sync_copy(acc.at[pl.ds(s * 640 + z * 40, 40)],
                            out.at[pl.ds(row_off + s * 640 + z * 40, 40)])

        plsc.subcore_barrier()


def _tc_spec(idx_map, shape):
    return pl.BlockSpec(shape, idx_map)


def _init_mm_body(head_ref, degp_ref, ent_ref, q_ref, w_ref, bias_ref,
                  out_ref):
    b = pl.program_id(0)
    i = pl.program_id(1)
    deg = (degp_ref[0] + degp_ref[1])[:, 0:1]
    hid = jnp.where(deg >= DEGREE, ent_ref[...],
                    jnp.float32(C0))
    nglob = i * BN + lax.broadcasted_iota(jnp.int32, (BN, 1), 0)
    is_head = nglob == head_ref[b]
    hid = jnp.where(is_head, q_ref[...], hid)
    out_ref[...] = (
        jnp.dot(hid, w_ref[...], preferred_element_type=jnp.float32)
        + bias_ref[...]
    )


def _init_mm(head, degp, ent_emb, query_emb, w, bias):
    return pl.pallas_call(
        _init_mm_body,
        grid=(B, NBLK),
        in_specs=[
            pl.BlockSpec(memory_space=pltpu.SMEM),
            _tc_spec(lambda b, i: (0, i, 0), (2, BN, 16)),
            _tc_spec(lambda b, i: (i, 0), (BN, H)),
            _tc_spec(lambda b, i: (0, 0), (1, H)),
            _tc_spec(lambda b, i: (0, 0), (H, H)),
            _tc_spec(lambda b, i: (0, 0), (1, H)),
        ],
        out_specs=_tc_spec(lambda b, i: (b * NBLK + i, 0), (BN, H)),
        out_shape=jax.ShapeDtypeStruct((B * N_PAD, H), jnp.float32),
    )(head, degp, ent_emb, query_emb, w, bias)


def _post_layer(upd, xe, wl, bl, g, beta, eps=1e-5):
    t = jnp.dot(upd, wl, preferred_element_type=jnp.float32) + bl + xe
    mu = jnp.mean(t, axis=-1, keepdims=True)
    var = jnp.mean((t - mu) ** 2, axis=-1, keepdims=True)
    return jnp.maximum(g * (t - mu) / jnp.sqrt(var + eps) + beta, 0.0)


def _mid_mm_body(upd_ref, xe_ref, wl_ref, bl_ref, g_ref, beta_ref, w2_ref,
                 b2_ref, out_ref):
    y = _post_layer(upd_ref[...], xe_ref[...], wl_ref[...], bl_ref[...],
                    g_ref[...], beta_ref[...])
    out_ref[...] = (
        jnp.dot(y, w2_ref[...], preferred_element_type=jnp.float32)
        + b2_ref[...]
    )


def _mid_mm(upd, xe, wl, bl, g, beta, w2, b2):
    blk = _tc_spec(lambda b, i: (b * NBLK + i, 0), (BN, H))
    par = _tc_spec(lambda b, i: (0, 0), (1, H))
    return pl.pallas_call(
        _mid_mm_body,
        grid=(B, NBLK),
        in_specs=[blk, blk, _tc_spec(lambda b, i: (0, 0), (H, H)), par, par,
                  par, _tc_spec(lambda b, i: (0, 0), (H, H)), par],
        out_specs=blk,
        out_shape=jax.ShapeDtypeStruct((B * N_PAD, H), jnp.float32),
    )(upd, xe, wl, bl, g, beta, w2, b2)


def _final_body(upd_ref, xe_ref, wl_ref, bl_ref, g_ref, beta_ref, wc_ref,
                bc_ref, out_ref):
    y = _post_layer(upd_ref[...], xe_ref[...], wl_ref[...], bl_ref[...],
                    g_ref[...], beta_ref[...])
    out_ref[...] = (
        jnp.dot(y, wc_ref[...], preferred_element_type=jnp.float32)
        + bc_ref[...]
    )


def _final_mm(upd, xe, wl, bl, g, beta, wc_pad, bc):
    blk = _tc_spec(lambda b, i: (b * NBLK + i, 0), (BN, H))
    par = _tc_spec(lambda b, i: (0, 0), (1, H))
    return pl.pallas_call(
        _final_body,
        grid=(B, NBLK),
        in_specs=[blk, blk, _tc_spec(lambda b, i: (0, 0), (H, H)), par, par,
                  par, _tc_spec(lambda b, i: (0, 0), (H, H)), par],
        out_specs=blk,
        out_shape=jax.ShapeDtypeStruct((B * N_PAD, H), jnp.float32),
    )(upd, xe, wl, bl, g, beta, wc_pad, bc)


def kernel(head, rel, edge_src, edge_rel, edge_dst, ent_emb, query_emb,
           rel_embs, W_ent, b_ent, W_lin, b_lin, ln_g, ln_b, W_cls, b_cls):
    npad = E_PAD - N_EDGES
    es = jnp.concatenate([edge_src, jnp.zeros((npad,), jnp.int32)])
    er = jnp.concatenate([edge_rel, jnp.zeros((npad,), jnp.int32)])
    ed = jnp.concatenate([edge_dst, jnp.full((npad,), N_ENT, jnp.int32)])

    degp = _degree_sc(er, ed)

    ent_p = jnp.concatenate(
        [ent_emb, jnp.zeros((N_PAD - N_ENT, H), jnp.float32)])
    xe = _init_mm(head, degp, ent_p, query_emb, W_ent[0],
                  b_ent[0].reshape(1, H))
    upd = _rspmm_sc(es, er, ed, rel_embs[0], xe)
    xe = _mid_mm(upd, xe, W_lin[0], b_lin[0].reshape(1, H),
                 ln_g[0].reshape(1, H), ln_b[0].reshape(1, H),
                 W_ent[1], b_ent[1].reshape(1, H))
    upd = _rspmm_sc(es, er, ed, rel_embs[1], xe)

    wc_pad = jnp.zeros((H, H), jnp.float32).at[:, 0].set(W_cls[:, 0])
    bc = jnp.zeros((1, H), jnp.float32).at[0, 0].set(b_cls[0])
    outf = _final_mm(upd, xe, W_lin[1], b_lin[1].reshape(1, H),
                     ln_g[1].reshape(1, H), ln_b[1].reshape(1, H),
                     wc_pad, bc)
    return outf.reshape(B, N_PAD, H)[:, :N_ENT, 0]


# speedup vs baseline: 5.2208x; 1.0137x over previous
import functools

import jax
import jax.numpy as jnp
import numpy as np
from jax import lax
from jax.experimental import pallas as pl
from jax.experimental.pallas import tpu as pltpu
from jax.experimental.pallas import tpu_sc as plsc

N_ENT = 10000
N_REL = 41
H = 128
N_EDGES = 160000
B = 4
DEGREE = 3

NW = 16
E_PAD = 163840
EPW = E_PAD // NW
K = 80
NCH = 10240 // K
N_PAD = 10240
NACC = N_PAD
BN = 320
NBLK = N_PAD // BN
C0 = float(1.0 / np.sqrt(H) - 0.5)

_mesh = plsc.VectorSubcoreMesh(core_axis_name="c", subcore_axis_name="s")


def _iota16():
    return lax.broadcasted_iota(jnp.int32, (16,), 0)


KD = E_PAD // 32


@functools.partial(
    pl.kernel,
    out_type=jax.ShapeDtypeStruct((2, N_PAD, 16), jnp.float32),
    mesh=_mesh,
    compiler_params=pltpu.CompilerParams(use_tc_tiling_on_sc=False),
    scratch_types=[
        pltpu.VMEM_SHARED((NACC, 16), jnp.float32),
        pltpu.VMEM((KD,), jnp.int32),
        pltpu.VMEM((KD,), jnp.int32),
        pltpu.VMEM((KD, 16), jnp.float32),
        pltpu.VMEM((64, 16), jnp.float32),
    ],
)
def _degree_sc(er, ed, out, acc, relb, dstb, rowsb, zbuf):
    c = lax.axis_index("c")
    s = lax.axis_index("s")
    wid = c * NW + s

    @pl.loop(0, 64)
    def _(i):
        zbuf[i, pl.ds(0, 16)] = jnp.zeros((16,), jnp.float32)

    for z in range(10):
        pltpu.sync_copy(zbuf, acc.at[pl.ds(s * 640 + z * 64, 64)])
    plsc.subcore_barrier()

    base = wid * KD
    pltpu.sync_copy(er.at[pl.ds(base, KD)], relb)
    pltpu.sync_copy(ed.at[pl.ds(base, KD)], dstb)

    @pl.loop(0, KD // 16)
    def _(i):
        r16 = relb[pl.ds(i * 16, 16)]
        v16 = jnp.where((r16 >= 1) & (r16 < (N_REL - 1) // 2),
                        jnp.float32(1.0), jnp.float32(0.0))
        for j in range(16):
            rowsb[i * 16 + j, pl.ds(0, 16)] = jnp.broadcast_to(v16[j], (16,))

    pltpu.sync_copy(rowsb, acc.at[dstb], add=True)
    plsc.subcore_barrier()

    @pl.loop(0, 16)
    def _(z):
        pltpu.sync_copy(acc.at[pl.ds(s * 640 + z * 40, 40)],
                        out.at[c, pl.ds(s * 640 + z * 40, 40), :])


@functools.partial(
    pl.kernel,
    out_type=jax.ShapeDtypeStruct((B * N_PAD, H), jnp.float32),
    mesh=_mesh,
    compiler_params=pltpu.CompilerParams(use_tc_tiling_on_sc=False),
    scratch_types=[
        pltpu.VMEM_SHARED((NACC, H), jnp.float32),
        pltpu.VMEM((K,), jnp.int32),
        pltpu.VMEM((K,), jnp.int32),
        pltpu.VMEM((K,), jnp.int32),
        pltpu.VMEM((K,), jnp.int32),
        pltpu.VMEM((K,), jnp.int32),
        pltpu.VMEM((K,), jnp.int32),
        pltpu.VMEM((K, H), jnp.float32),
        pltpu.VMEM((K, H), jnp.float32),
        pltpu.VMEM((K, H), jnp.float32),
        pltpu.VMEM((K, H), jnp.float32),
        pltpu.VMEM((K,), jnp.int32),
        pltpu.VMEM((K,), jnp.int32),
        pltpu.VMEM((16, H), jnp.float32),
        pltpu.SemaphoreType.DMA,
        pltpu.SemaphoreType.DMA,
        pltpu.SemaphoreType.DMA,
        pltpu.SemaphoreType.DMA,
        pltpu.SemaphoreType.DMA,
        pltpu.SemaphoreType.DMA,
        pltpu.SemaphoreType.DMA,
        pltpu.SemaphoreType.DMA,
        pltpu.SemaphoreType.DMA,
        pltpu.SemaphoreType.DMA,
        pltpu.SemaphoreType.DMA,
        pltpu.SemaphoreType.DMA,
    ],
)
def _rspmm_sc(es, er, ed, rel_t, xe, out, acc, srcb0, srcb1, relb0, relb1,
              dstb0, dstb1, rows0, rows1, rrows0, rrows1, sidx0, sidx1,
              zbuf, sg0, sg1, sq0, sq1, ss0, ss1, sa0, sa1, sb0, sb1,
              sc0, sc1):
    c = lax.axis_index("c")
    s = lax.axis_index("s")
    srcb = (srcb0, srcb1)
    relb = (relb0, relb1)
    dstb = (dstb0, dstb1)
    rows = (rows0, rows1)
    rrows = (rrows0, rrows1)
    sidx = (sidx0, sidx1)
    sg = (sg0, sg1)
    sq = (sq0, sq1)
    ss = (ss0, ss1)
    sa = (sa0, sa1)
    sb = (sb0, sb1)
    sc = (sc0, sc1)

    @pl.loop(0, 16)
    def _(i):
        for h in range(H // 16):
            zbuf[i, pl.ds(h * 16, 16)] = jnp.zeros((16,), jnp.float32)

    for b in range(2):
        bidx = c * 2 + b
        row_off = bidx * N_PAD

        for z in range(40):
            pltpu.sync_copy(zbuf, acc.at[pl.ds(s * 640 + z * 16, 16)])

        plsc.subcore_barrier()

        def start_ids(i, ch):
            ebase = s * EPW + ch * K
            pltpu.async_copy(es.at[pl.ds(ebase, K)], srcb[i], sa[i])
            pltpu.async_copy(er.at[pl.ds(ebase, K)], relb[i], sb[i])
            pltpu.async_copy(ed.at[pl.ds(ebase, K)], dstb[i], sc[i])

        def wait_ids_start_gathers(i, ch):
            ebase = s * EPW + ch * K
            pltpu.make_async_copy(es.at[pl.ds(ebase, K)], srcb[i],
                                  sa[i]).wait()
            pltpu.make_async_copy(er.at[pl.ds(ebase, K)], relb[i],
                                  sb[i]).wait()
            pltpu.make_async_copy(ed.at[pl.ds(ebase, K)], dstb[i],
                                  sc[i]).wait()

            @pl.loop(0, K // 16)
            def _(j):
                srcb[i][pl.ds(j * 16, 16)] = (
                    srcb[i][pl.ds(j * 16, 16)] + row_off)

            pltpu.async_copy(xe.at[srcb[i]], rows[i], sg[i])
            pltpu.async_copy(rel_t.at[relb[i]], rrows[i], sq[i])

        def wait_gathers(i):
            pltpu.make_async_copy(xe.at[srcb[i]], rows[i], sg[i]).wait()
            pltpu.make_async_copy(rel_t.at[relb[i]], rrows[i], sq[i]).wait()

        def multiply(i):
            @pl.loop(0, K // 2)
            def _(k2):
                for u in range(2):
                    k = k2 * 2 + u
                    for h in range(H // 16):
                        rows[i][k, pl.ds(h * 16, 16)] = (
                            rows[i][k, pl.ds(h * 16, 16)]
                            * rrows[i][k, pl.ds(h * 16, 16)]
                        )

        def snapshot_dst(i):
            @pl.loop(0, K // 16)
            def _(j):
                sidx[i][pl.ds(j * 16, 16)] = dstb[i][pl.ds(j * 16, 16)]

        start_ids(0, 0)
        start_ids(1, 1)
        wait_ids_start_gathers(0, 0)
        wait_ids_start_gathers(1, 1)

        @pl.loop(0, NCH // 2)
        def _(gg):
            for i in range(2):
                wait_gathers(i)
                multiply(i)
                snapshot_dst(i)
                pltpu.async_copy(rows[i], acc.at[sidx[i]], ss[i], add=True)

                @pl.when(gg < NCH // 2 - 1)
                def _():
                    start_ids(i, 2 * gg + 2 + i)

            for i in range(2):
                pltpu.make_async_copy(rows[i], acc.at[sidx[i]],
                                      ss[i]).wait()

                @pl.when(gg < NCH // 2 - 1)
                def _():
                    wait_ids_start_gathers(i, 2 * gg + 2 + i)

        plsc.subcore_barrier()

        @pl.loop(0, 16)
        def _(z):
            pltpu.sync_copy(acc.at[pl.ds(s * 640 + z * 40, 40)],
                            out.at[pl.ds(row_off + s * 640 + z * 40, 40)])

        plsc.subcore_barrier()


def _tc_spec(idx_map, shape):
    return pl.BlockSpec(shape, idx_map)


def _init_mm_body(head_ref, degp_ref, ent_ref, q_ref, w_ref, bias_ref,
                  out_ref):
    b = pl.program_id(0)
    i = pl.program_id(1)
    deg = (degp_ref[0] + degp_ref[1])[:, 0:1]
    hid = jnp.where(deg >= DEGREE, ent_ref[...],
                    jnp.float32(C0))
    nglob = i * BN + lax.broadcasted_iota(jnp.int32, (BN, 1), 0)
    is_head = nglob == head_ref[b]
    hid = jnp.where(is_head, q_ref[...], hid)
    out_ref[...] = (
        jnp.dot(hid, w_ref[...], preferred_element_type=jnp.float32)
        + bias_ref[...]
    )


def _init_mm(head, degp, ent_emb, query_emb, w, bias):
    return pl.pallas_call(
        _init_mm_body,
        grid=(B, NBLK),
        in_specs=[
            pl.BlockSpec(memory_space=pltpu.SMEM),
            _tc_spec(lambda b, i: (0, i, 0), (2, BN, 16)),
            _tc_spec(lambda b, i: (i, 0), (BN, H)),
            _tc_spec(lambda b, i: (0, 0), (1, H)),
            _tc_spec(lambda b, i: (0, 0), (H, H)),
            _tc_spec(lambda b, i: (0, 0), (1, H)),
        ],
        out_specs=_tc_spec(lambda b, i: (b * NBLK + i, 0), (BN, H)),
        out_shape=jax.ShapeDtypeStruct((B * N_PAD, H), jnp.float32),
    )(head, degp, ent_emb, query_emb, w, bias)


def _post_layer(upd, xe, wl, bl, g, beta, eps=1e-5):
    t = jnp.dot(upd, wl, preferred_element_type=jnp.float32) + bl + xe
    mu = jnp.mean(t, axis=-1, keepdims=True)
    var = jnp.mean((t - mu) ** 2, axis=-1, keepdims=True)
    return jnp.maximum(g * (t - mu) / jnp.sqrt(var + eps) + beta, 0.0)


def _mid_mm_body(upd_ref, xe_ref, wl_ref, bl_ref, g_ref, beta_ref, w2_ref,
                 b2_ref, out_ref):
    y = _post_layer(upd_ref[...], xe_ref[...], wl_ref[...], bl_ref[...],
                    g_ref[...], beta_ref[...])
    out_ref[...] = (
        jnp.dot(y, w2_ref[...], preferred_element_type=jnp.float32)
        + b2_ref[...]
    )


def _mid_mm(upd, xe, wl, bl, g, beta, w2, b2):
    blk = _tc_spec(lambda b, i: (b * NBLK + i, 0), (BN, H))
    par = _tc_spec(lambda b, i: (0, 0), (1, H))
    return pl.pallas_call(
        _mid_mm_body,
        grid=(B, NBLK),
        in_specs=[blk, blk, _tc_spec(lambda b, i: (0, 0), (H, H)), par, par,
                  par, _tc_spec(lambda b, i: (0, 0), (H, H)), par],
        out_specs=blk,
        out_shape=jax.ShapeDtypeStruct((B * N_PAD, H), jnp.float32),
    )(upd, xe, wl, bl, g, beta, w2, b2)


def _final_body(upd_ref, xe_ref, wl_ref, bl_ref, g_ref, beta_ref, wc_ref,
                bc_ref, out_ref):
    y = _post_layer(upd_ref[...], xe_ref[...], wl_ref[...], bl_ref[...],
                    g_ref[...], beta_ref[...])
    out_ref[...] = (
        jnp.dot(y, wc_ref[...], preferred_element_type=jnp.float32)
        + bc_ref[...]
    )


def _final_mm(upd, xe, wl, bl, g, beta, wc_pad, bc):
    blk = _tc_spec(lambda b, i: (b * NBLK + i, 0), (BN, H))
    par = _tc_spec(lambda b, i: (0, 0), (1, H))
    return pl.pallas_call(
        _final_body,
        grid=(B, NBLK),
        in_specs=[blk, blk, _tc_spec(lambda b, i: (0, 0), (H, H)), par, par,
                  par, _tc_spec(lambda b, i: (0, 0), (H, H)), par],
        out_specs=blk,
        out_shape=jax.ShapeDtypeStruct((B * N_PAD, H), jnp.float32),
    )(upd, xe, wl, bl, g, beta, wc_pad, bc)


def kernel(head, rel, edge_src, edge_rel, edge_dst, ent_emb, query_emb,
           rel_embs, W_ent, b_ent, W_lin, b_lin, ln_g, ln_b, W_cls, b_cls):
    npad = E_PAD - N_EDGES
    es = jnp.concatenate([edge_src, jnp.zeros((npad,), jnp.int32)])
    er = jnp.concatenate([edge_rel, jnp.zeros((npad,), jnp.int32)])
    ed = jnp.concatenate([edge_dst, jnp.full((npad,), N_ENT, jnp.int32)])

    degp = _degree_sc(er, ed)

    ent_p = jnp.concatenate(
        [ent_emb, jnp.zeros((N_PAD - N_ENT, H), jnp.float32)])
    xe = _init_mm(head, degp, ent_p, query_emb, W_ent[0],
                  b_ent[0].reshape(1, H))
    upd = _rspmm_sc(es, er, ed, rel_embs[0], xe)
    xe = _mid_mm(upd, xe, W_lin[0], b_lin[0].reshape(1, H),
                 ln_g[0].reshape(1, H), ln_b[0].reshape(1, H),
                 W_ent[1], b_ent[1].reshape(1, H))
    upd = _rspmm_sc(es, er, ed, rel_embs[1], xe)

    wc_pad = jnp.zeros((H, H), jnp.float32).at[:, 0].set(W_cls[:, 0])
    bc = jnp.zeros((1, H), jnp.float32).at[0, 0].set(b_cls[0])
    outf = _final_mm(upd, xe, W_lin[1], b_lin[1].reshape(1, H),
                     ln_g[1].reshape(1, H), ln_b[1].reshape(1, H),
                     wc_pad, bc)
    return outf.reshape(B, N_PAD, H)[:, :N_ENT, 0]
